# baseline (speedup 1.0000x reference)
import functools

import jax
import jax.numpy as jnp
from jax import lax
from jax.experimental import pallas as pl
from jax.experimental.pallas import tpu as pltpu
from jax.experimental.pallas import tpu_sc as plsc

NN = 10000
EE = 320000
DD = 128
NC = 2
NS = 16
NW = NC * NS
EPT = EE // NW
CK = 40
NCH = EPT // CK
NROW = EE // CK
NPAD = 10240
RPT = NPAD // NS
RPT2 = NN // NS

_mesh = plsc.VectorSubcoreMesh(
    core_axis_name="c", subcore_axis_name="s", num_cores=NC, num_subcores=NS
)
_sc_params = pltpu.CompilerParams(
    use_tc_tiling_on_sc=False, needs_layout_passes=False
)


@functools.partial(
    pl.kernel,
    out_type=jax.ShapeDtypeStruct((NC, NPAD), jnp.float32),
    mesh=_mesh,
    compiler_params=_sc_params,
    scratch_types=[
        pltpu.VMEM((NCH, CK), jnp.int32),
        pltpu.VMEM((NCH, CK), jnp.float32),
        pltpu.VMEM((RPT,), jnp.float32),
        pltpu.VMEM_SHARED((NPAD,), jnp.float32),
    ],
)
def _deg_kernel(dst_hbm, w_hbm, out_hbm, dst_v, w_v, zer_v, acc_sh):
  cid = lax.axis_index("c")
  sid = lax.axis_index("s")
  wid = cid * NS + sid
  zvec = jnp.zeros((16,), jnp.float32)

  def _zero(i, carry):
    zer_v[pl.ds(i * 16, 16)] = zvec
    return carry

  lax.fori_loop(0, RPT // 16, _zero, 0)
  pltpu.sync_copy(zer_v, acc_sh.at[pl.ds(sid * RPT, RPT)])
  plsc.subcore_barrier()

  r0 = wid * NCH
  pltpu.sync_copy(dst_hbm.at[pl.ds(r0, NCH)], dst_v)
  pltpu.sync_copy(w_hbm.at[pl.ds(r0, NCH)], w_v)

  def _scatter(g, carry):
    pltpu.sync_copy(w_v.at[g], acc_sh.at[dst_v.at[g]], add=True)
    return carry

  lax.fori_loop(0, NCH, _scatter, 0)
  plsc.subcore_barrier()
  pltpu.sync_copy(
      acc_sh.at[pl.ds(sid * RPT, RPT)], out_hbm.at[cid, pl.ds(sid * RPT, RPT)]
  )


def _tc1_body(x_ref, w1_ref, degp2_ref, hs_ref, dinv_ref):
  h = jnp.dot(x_ref[...], w1_ref[...], preferred_element_type=jnp.float32)
  deg = degp2_ref[0] + degp2_ref[1] + 1.0
  dinv = jnp.where(deg > 0, lax.rsqrt(deg), 0.0)
  dinv_ref[...] = dinv
  hs_ref[...] = h * dinv


def _tc1(x, w1, degp2):
  return pl.pallas_call(
      _tc1_body,
      out_shape=[
          jax.ShapeDtypeStruct((NN, DD), jnp.float32),
          jax.ShapeDtypeStruct((NN, 1), jnp.float32),
      ],
  )(x, w1, degp2)


@functools.partial(
    pl.kernel,
    out_type=jax.ShapeDtypeStruct((NC, NN, DD), jnp.float32),
    mesh=_mesh,
    compiler_params=_sc_params,
    scratch_types=[
        pltpu.VMEM((NCH, CK), jnp.int32),
        pltpu.VMEM((NCH, CK), jnp.int32),
        pltpu.VMEM((NCH, CK), jnp.float32),
        pltpu.VMEM((CK, DD), jnp.float32),
        pltpu.VMEM((CK, DD), jnp.float32),
        pltpu.VMEM_SHARED((NN, DD), jnp.float32),
        pltpu.SemaphoreType.DMA,
        pltpu.SemaphoreType.DMA,
        pltpu.SemaphoreType.DMA,
        pltpu.SemaphoreType.DMA,
    ],
)
def _conv_kernel(
    src_hbm, dst_hbm, w_hbm, hs_hbm, out_hbm,
    src_v, dst_v, w_v, rows_a, rows_b, acc_sh,
    gsa, gsb, ssa, ssb,
):
  cid = lax.axis_index("c")
  sid = lax.axis_index("s")
  wid = cid * NS + sid
  zvec = jnp.zeros((16,), jnp.float32)

  def _zero_rows(i, carry):
    for j in range(DD // 16):
      rows_a[i, pl.ds(j * 16, 16)] = zvec
    return carry

  lax.fori_loop(0, CK, _zero_rows, 0)

  def _zero_acc(k, carry):
    pltpu.sync_copy(rows_a, acc_sh.at[pl.ds(sid * RPT2 + k * CK, CK)])
    return carry

  lax.fori_loop(0, RPT2 // CK, _zero_acc, 0)
  pltpu.sync_copy(
      rows_a.at[pl.ds(0, RPT2 % CK)],
      acc_sh.at[pl.ds(sid * RPT2 + (RPT2 // CK) * CK, RPT2 % CK)],
  )
  plsc.subcore_barrier()

  r0 = wid * NCH
  pltpu.sync_copy(src_hbm.at[pl.ds(r0, NCH)], src_v)
  pltpu.sync_copy(dst_hbm.at[pl.ds(r0, NCH)], dst_v)
  pltpu.sync_copy(w_hbm.at[pl.ds(r0, NCH)], w_v)

  def _gather(g, rows, sem):
    pltpu.async_copy(hs_hbm.at[src_v.at[g]], rows, sem)

  def _wait_gather(rows, sem):
    pltpu.make_async_copy(hs_hbm.at[src_v.at[0]], rows, sem).wait()

  def _process(g, rows):
    @plsc.parallel_loop(0, CK // 16, step=1)
    def _scale(eb):
      w16 = w_v[g, pl.ds(eb * 16, 16)]
      for k in range(16):
        c = w16[k]
        e = eb * 16 + k
        for j in range(DD // 16):
          sl = pl.ds(j * 16, 16)
          rows[e, sl] = rows[e, sl] * c

  def _scatter(g, rows, sem):
    pltpu.async_copy(rows, acc_sh.at[dst_v.at[g]], sem, add=True)

  def _wait_scatter(rows, sem):
    pltpu.make_async_copy(rows, acc_sh.at[dst_v.at[0]], sem).wait()

  _gather(0, rows_a, gsa)
  _gather(1, rows_b, gsb)

  def _loop(it, carry):
    g0 = 2 * it
    _wait_gather(rows_a, gsa)
    _process(g0, rows_a)
    _scatter(g0, rows_a, ssa)
    _wait_gather(rows_b, gsb)
    _process(g0 + 1, rows_b)
    _scatter(g0 + 1, rows_b, ssb)
    _wait_scatter(rows_a, ssa)
    _gather(jnp.minimum(g0 + 2, NCH - 1), rows_a, gsa)
    _wait_scatter(rows_b, ssb)
    _gather(jnp.minimum(g0 + 3, NCH - 1), rows_b, gsb)
    return carry

  lax.fori_loop(0, (NCH - 1) // 2, _loop, 0)

  _wait_gather(rows_a, gsa)
  _process(NCH - 1, rows_a)
  pltpu.sync_copy(rows_a, acc_sh.at[dst_v.at[NCH - 1]], add=True)
  _wait_gather(rows_b, gsb)

  plsc.subcore_barrier()
  pltpu.sync_copy(
      acc_sh.at[pl.ds(sid * RPT2, RPT2)],
      out_hbm.at[cid, pl.ds(sid * RPT2, RPT2)],
  )


def _tc2_body(p_ref, hs_ref, dinv_ref, b1_ref, w2_ref, emb_ref, ts_ref):
  dinv = dinv_ref[...]
  out1 = dinv * (p_ref[0] + p_ref[1] + hs_ref[...]) + b1_ref[...]
  emb = jnp.where(out1 > 0, out1, jnp.exp(out1) - 1.0)
  emb_ref[...] = emb
  t = jnp.dot(emb, w2_ref[...], preferred_element_type=jnp.float32)
  ts_ref[...] = dinv * t


def _tc2(p, hs, dinv2d, b1, w2):
  return pl.pallas_call(
      _tc2_body,
      out_shape=[
          jax.ShapeDtypeStruct((NN, DD), jnp.float32),
          jax.ShapeDtypeStruct((NN, 1), jnp.float32),
      ],
  )(p, hs, dinv2d, b1, w2)


@functools.partial(
    pl.kernel,
    out_type=jax.ShapeDtypeStruct((NC, NPAD), jnp.float32),
    mesh=_mesh,
    compiler_params=_sc_params,
    scratch_types=[
        pltpu.VMEM((NCH, CK), jnp.int32),
        pltpu.VMEM((NCH, CK), jnp.int32),
        pltpu.VMEM((NCH, CK), jnp.float32),
        pltpu.VMEM((NN,), jnp.float32),
        pltpu.VMEM((CK,), jnp.float32),
        pltpu.VMEM((RPT,), jnp.float32),
        pltpu.VMEM_SHARED((NPAD,), jnp.float32),
    ],
)
def _l2_kernel(
    src_hbm, dst_hbm, w_hbm, ts_hbm, out_hbm,
    src_v, dst_v, w_v, ts_v, vals_v, zer_v, acc_sh,
):
  cid = lax.axis_index("c")
  sid = lax.axis_index("s")
  wid = cid * NS + sid
  zvec = jnp.zeros((16,), jnp.float32)

  def _zero(i, carry):
    zer_v[pl.ds(i * 16, 16)] = zvec
    return carry

  lax.fori_loop(0, RPT // 16, _zero, 0)
  pltpu.sync_copy(zer_v, acc_sh.at[pl.ds(sid * RPT, RPT)])
  plsc.subcore_barrier()

  r0 = wid * NCH
  pltpu.sync_copy(src_hbm.at[pl.ds(r0, NCH)], src_v)
  pltpu.sync_copy(dst_hbm.at[pl.ds(r0, NCH)], dst_v)
  pltpu.sync_copy(w_hbm.at[pl.ds(r0, NCH)], w_v)
  pltpu.sync_copy(ts_hbm, ts_v)

  def _chunk(g, carry):
    for i in range(CK // 16):
      sl = pl.ds(i * 16, 16)
      s16 = src_v[g, sl]
      ts = plsc.load_gather(ts_v, [s16])
      vals_v[sl] = w_v[g, sl] * ts
    pltpu.sync_copy(vals_v, acc_sh.at[dst_v.at[g]], add=True)
    return carry

  lax.fori_loop(0, NCH, _chunk, 0)
  plsc.subcore_barrier()
  pltpu.sync_copy(
      acc_sh.at[pl.ds(sid * RPT, RPT)], out_hbm.at[cid, pl.ds(sid * RPT, RPT)]
  )


def _tc3_body(q_ref, ts1_ref, dinv1_ref, b2_ref, out_ref):
  d = dinv1_ref[...]
  z = d * (q_ref[0, 0:NN] + q_ref[1, 0:NN] + ts1_ref[...]) + b2_ref[...]
  out_ref[...] = 1.0 / (1.0 + jnp.exp(-z))


def _tc3(q, ts1, dinv1, b2):
  return pl.pallas_call(
      _tc3_body,
      out_shape=jax.ShapeDtypeStruct((NN,), jnp.float32),
  )(q, ts1, dinv1, b2)


def kernel(x, edge_index, edge_attr, W1, b1, W2, b2):
  src2 = edge_index[0].reshape(NROW, CK)
  dst2 = edge_index[1].reshape(NROW, CK)
  w2d = edge_attr.reshape(NROW, CK)

  degp = _deg_kernel(dst2, w2d)
  degp2 = degp[:, :NN].reshape(NC, NN, 1)
  hs, dinv2d = _tc1(x, W1, degp2)
  p = _conv_kernel(src2, dst2, w2d, hs)
  emb, ts = _tc2(p, hs, dinv2d, b1, W2)
  q = _l2_kernel(src2, dst2, w2d, ts.reshape(NN))
  out1 = _tc3(q, ts.reshape(NN), dinv2d.reshape(NN), b2)
  return (out1.reshape(NN, 1), emb)


# speedup vs baseline: 1.2598x; 1.2598x over previous
import functools

import jax
import jax.numpy as jnp
from jax import lax
from jax.experimental import pallas as pl
from jax.experimental.pallas import tpu as pltpu
from jax.experimental.pallas import tpu_sc as plsc

NN = 10000
EE = 320000
DD = 128
NC = 2
NS = 16
NW = NC * NS
EPT = EE // NW
CK = 80
NCH = EPT // CK
NROW = EE // CK
NPAD = 10240
RPT = NPAD // NS
RPT2 = NN // NS

_mesh = plsc.VectorSubcoreMesh(
    core_axis_name="c", subcore_axis_name="s", num_cores=NC, num_subcores=NS
)
_sc_params = pltpu.CompilerParams(
    use_tc_tiling_on_sc=False, needs_layout_passes=False
)


@functools.partial(
    pl.kernel,
    out_type=jax.ShapeDtypeStruct((NC, NPAD), jnp.float32),
    mesh=_mesh,
    compiler_params=_sc_params,
    scratch_types=[
        pltpu.VMEM((NCH, CK), jnp.int32),
        pltpu.VMEM((NCH, CK), jnp.float32),
        pltpu.VMEM((RPT,), jnp.float32),
        pltpu.VMEM_SHARED((NPAD,), jnp.float32),
    ],
)
def _deg_kernel(dst_hbm, w_hbm, out_hbm, dst_v, w_v, zer_v, acc_sh):
  cid = lax.axis_index("c")
  sid = lax.axis_index("s")
  wid = cid * NS + sid
  zvec = jnp.zeros((16,), jnp.float32)

  def _zero(i, carry):
    zer_v[pl.ds(i * 16, 16)] = zvec
    return carry

  lax.fori_loop(0, RPT // 16, _zero, 0)
  pltpu.sync_copy(zer_v, acc_sh.at[pl.ds(sid * RPT, RPT)])
  plsc.subcore_barrier()

  r0 = wid * NCH
  pltpu.sync_copy(dst_hbm.at[pl.ds(r0, NCH)], dst_v)
  pltpu.sync_copy(w_hbm.at[pl.ds(r0, NCH)], w_v)

  def _scatter(g, carry):
    pltpu.sync_copy(w_v.at[g], acc_sh.at[dst_v.at[g]], add=True)
    return carry

  lax.fori_loop(0, NCH, _scatter, 0)
  plsc.subcore_barrier()
  pltpu.sync_copy(
      acc_sh.at[pl.ds(sid * RPT, RPT)], out_hbm.at[cid, pl.ds(sid * RPT, RPT)]
  )


def _tc1_body(x_ref, w1_ref, degp2_ref, hs_ref, dinv_ref):
  h = jnp.dot(x_ref[...], w1_ref[...], preferred_element_type=jnp.float32)
  deg = degp2_ref[0] + degp2_ref[1] + 1.0
  dinv = jnp.where(deg > 0, lax.rsqrt(deg), 0.0)
  dinv_ref[...] = dinv
  hs_ref[...] = h * dinv


def _tc1(x, w1, degp2):
  return pl.pallas_call(
      _tc1_body,
      out_shape=[
          jax.ShapeDtypeStruct((NN, DD), jnp.float32),
          jax.ShapeDtypeStruct((NN, 1), jnp.float32),
      ],
  )(x, w1, degp2)


GR = DD // 32


@functools.partial(
    pl.kernel,
    out_type=jax.ShapeDtypeStruct((NC, NN, DD), jnp.float32),
    mesh=_mesh,
    compiler_params=_sc_params,
    scratch_types=[
        pltpu.VMEM((NCH, CK), jnp.int32),
        pltpu.VMEM((NCH, CK), jnp.float32),
        pltpu.VMEM((CK, DD), jnp.bfloat16),
        pltpu.VMEM((CK, DD), jnp.bfloat16),
        pltpu.VMEM((CK, DD), jnp.bfloat16),
        pltpu.VMEM((CK, DD), jnp.float32),
        pltpu.VMEM((CK,), jnp.int32),
        pltpu.VMEM((CK,), jnp.int32),
        pltpu.VMEM((CK,), jnp.int32),
        pltpu.VMEM((CK,), jnp.int32),
        pltpu.VMEM_SHARED((NN, DD), jnp.float32),
        pltpu.SemaphoreType.DMA,
        pltpu.SemaphoreType.DMA,
        pltpu.SemaphoreType.DMA,
        pltpu.SemaphoreType.DMA,
    ],
)
def _conv_kernel(
    pk_hbm, w_hbm, hsb_hbm, out_hbm,
    pk_v, w_v, rb0, rb1, rb2, stg, si0, si1, si2, didx, acc_sh,
    gs0, gs1, gs2, ssem,
):
  cid = lax.axis_index("c")
  sid = lax.axis_index("s")
  wid = cid * NS + sid
  zvec = jnp.zeros((16,), jnp.float32)
  zivec = jnp.zeros((16,), jnp.int32)

  rbufs = (rb0, rb1, rb2)
  sibufs = (si0, si1, si2)
  gsems = (gs0, gs1, gs2)

  def _zero_rows(i, carry):
    for j in range(DD // 16):
      stg[i, pl.ds(j * 16, 16)] = zvec
    return carry

  lax.fori_loop(0, CK, _zero_rows, 0)

  def _zero_acc(k, carry):
    pltpu.sync_copy(stg, acc_sh.at[pl.ds(sid * RPT2 + k * CK, CK)])
    return carry

  lax.fori_loop(0, RPT2 // CK, _zero_acc, 0)
  pltpu.sync_copy(
      stg.at[pl.ds(0, RPT2 % CK)],
      acc_sh.at[pl.ds(sid * RPT2 + (RPT2 // CK) * CK, RPT2 % CK)],
  )
  plsc.subcore_barrier()

  r0 = wid * NCH
  pltpu.sync_copy(pk_hbm.at[pl.ds(r0, NCH)], pk_v)
  pltpu.sync_copy(w_hbm.at[pl.ds(r0, NCH)], w_v)

  def _unpack_src(g, sib):
    for i in range(CK // 16):
      sl = pl.ds(i * 16, 16)
      sib[sl] = pk_v[g, sl] & 16383

  def _gather(sib, rb, sem):
    pltpu.async_copy(hsb_hbm.at[sib], rb, sem)

  def _wait_gather(rb, sem):
    pltpu.make_async_copy(hsb_hbm.at[si0], rb, sem).wait()

  def _process(g, rb):
    for i in range(CK // 16):
      sl = pl.ds(i * 16, 16)
      didx[sl] = pk_v[g, sl] >> 14

    g16 = jnp.full((16,), g, jnp.int32)

    @plsc.parallel_loop(0, CK, step=1, unroll=2)
    def _scale(e):
      cs = plsc.load_gather(w_v, [g16, jnp.full((16,), e, jnp.int32)])
      for j in range(GR):
        v = plsc.bitcast(rb[e, pl.ds(32 * j, 32)], jnp.int32)
        lo = plsc.bitcast(v << 16, jnp.float32)
        hi = plsc.bitcast(v & jnp.int32(-65536), jnp.float32)
        stg[e, pl.ds(32 * j, 16)] = lo * cs
        stg[e, pl.ds(32 * j + 16, 16)] = hi * cs

  def _wait_scatter():
    pltpu.make_async_copy(stg, acc_sh.at[didx], ssem).wait()

  for i in range(CK // 16):
    didx[pl.ds(i * 16, 16)] = zivec
  pltpu.async_copy(stg, acc_sh.at[didx], ssem, add=True)

  for b in range(3):
    _unpack_src(b, sibufs[b])
    _gather(sibufs[b], rbufs[b], gsems[b])

  def _step(c, b):
    _wait_gather(rbufs[b], gsems[b])
    _wait_scatter()
    _process(c, rbufs[b])
    pltpu.async_copy(stg, acc_sh.at[didx], ssem, add=True)
    cn = jnp.minimum(c + 3, NCH - 1)
    _unpack_src(cn, sibufs[b])
    _gather(sibufs[b], rbufs[b], gsems[b])

  def _loop(it, carry):
    c = 3 * it
    _step(c, 0)
    _step(c + 1, 1)
    _step(c + 2, 2)
    return carry

  lax.fori_loop(0, NCH // 3, _loop, 0)

  def _tail(c, b):
    _wait_gather(rbufs[b], gsems[b])
    _wait_scatter()
    _process(c, rbufs[b])
    pltpu.async_copy(stg, acc_sh.at[didx], ssem, add=True)

  _tail(NCH - 2, 0)
  _tail(NCH - 1, 1)
  _wait_gather(rbufs[2], gsems[2])
  _wait_scatter()

  plsc.subcore_barrier()
  pltpu.sync_copy(
      acc_sh.at[pl.ds(sid * RPT2, RPT2)],
      out_hbm.at[cid, pl.ds(sid * RPT2, RPT2)],
  )


def _tc2_body(p_ref, hs_ref, dinv_ref, b1_ref, w2_ref, emb_ref, ts_ref):
  dinv = dinv_ref[...]
  out1 = dinv * (p_ref[0] + p_ref[1] + hs_ref[...]) + b1_ref[...]
  emb = jnp.where(out1 > 0, out1, jnp.exp(out1) - 1.0)
  emb_ref[...] = emb
  t = jnp.dot(emb, w2_ref[...], preferred_element_type=jnp.float32)
  ts_ref[...] = dinv * t


def _tc2(p, hs, dinv2d, b1, w2):
  return pl.pallas_call(
      _tc2_body,
      out_shape=[
          jax.ShapeDtypeStruct((NN, DD), jnp.float32),
          jax.ShapeDtypeStruct((NN, 1), jnp.float32),
      ],
  )(p, hs, dinv2d, b1, w2)


@functools.partial(
    pl.kernel,
    out_type=jax.ShapeDtypeStruct((NC, NPAD), jnp.float32),
    mesh=_mesh,
    compiler_params=_sc_params,
    scratch_types=[
        pltpu.VMEM((NCH, CK), jnp.int32),
        pltpu.VMEM((NCH, CK), jnp.int32),
        pltpu.VMEM((NCH, CK), jnp.float32),
        pltpu.VMEM((NN,), jnp.float32),
        pltpu.VMEM((CK,), jnp.float32),
        pltpu.VMEM((RPT,), jnp.float32),
        pltpu.VMEM_SHARED((NPAD,), jnp.float32),
    ],
)
def _l2_kernel(
    src_hbm, dst_hbm, w_hbm, ts_hbm, out_hbm,
    src_v, dst_v, w_v, ts_v, vals_v, zer_v, acc_sh,
):
  cid = lax.axis_index("c")
  sid = lax.axis_index("s")
  wid = cid * NS + sid
  zvec = jnp.zeros((16,), jnp.float32)

  def _zero(i, carry):
    zer_v[pl.ds(i * 16, 16)] = zvec
    return carry

  lax.fori_loop(0, RPT // 16, _zero, 0)
  pltpu.sync_copy(zer_v, acc_sh.at[pl.ds(sid * RPT, RPT)])
  plsc.subcore_barrier()

  r0 = wid * NCH
  pltpu.sync_copy(src_hbm.at[pl.ds(r0, NCH)], src_v)
  pltpu.sync_copy(dst_hbm.at[pl.ds(r0, NCH)], dst_v)
  pltpu.sync_copy(w_hbm.at[pl.ds(r0, NCH)], w_v)
  pltpu.sync_copy(ts_hbm, ts_v)

  def _chunk(g, carry):
    for i in range(CK // 16):
      sl = pl.ds(i * 16, 16)
      s16 = src_v[g, sl]
      ts = plsc.load_gather(ts_v, [s16])
      vals_v[sl] = w_v[g, sl] * ts
    pltpu.sync_copy(vals_v, acc_sh.at[dst_v.at[g]], add=True)
    return carry

  lax.fori_loop(0, NCH, _chunk, 0)
  plsc.subcore_barrier()
  pltpu.sync_copy(
      acc_sh.at[pl.ds(sid * RPT, RPT)], out_hbm.at[cid, pl.ds(sid * RPT, RPT)]
  )


def _tc3_body(q_ref, ts1_ref, dinv1_ref, b2_ref, out_ref):
  d = dinv1_ref[...]
  z = d * (q_ref[0, 0:NN] + q_ref[1, 0:NN] + ts1_ref[...]) + b2_ref[...]
  out_ref[...] = 1.0 / (1.0 + jnp.exp(-z))


def _tc3(q, ts1, dinv1, b2):
  return pl.pallas_call(
      _tc3_body,
      out_shape=jax.ShapeDtypeStruct((NN,), jnp.float32),
  )(q, ts1, dinv1, b2)


_COLPERM = []
for _g in range(DD // 32):
  for _m in range(16):
    _COLPERM.append(32 * _g + _m)
    _COLPERM.append(32 * _g + 16 + _m)
_COLPERM = tuple(_COLPERM)


def kernel(x, edge_index, edge_attr, W1, b1, W2, b2):
  src2 = edge_index[0].reshape(NROW, CK)
  dst2 = edge_index[1].reshape(NROW, CK)
  w2d = edge_attr.reshape(NROW, CK)
  pk2 = src2 | (dst2 << 14)

  degp = _deg_kernel(dst2, w2d)
  degp2 = degp[:, :NN].reshape(NC, NN, 1)
  hs, dinv2d = _tc1(x, W1, degp2)
  hsb = hs[:, _COLPERM].astype(jnp.bfloat16)
  p = _conv_kernel(pk2, w2d, hsb)
  emb, ts = _tc2(p, hs, dinv2d, b1, W2)
  q = _l2_kernel(src2, dst2, w2d, ts.reshape(NN))
  out1 = _tc3(q, ts.reshape(NN), dinv2d.reshape(NN), b2)
  return (out1.reshape(NN, 1), emb)


# speedup vs baseline: 1.2996x; 1.0316x over previous
import functools

import jax
import jax.numpy as jnp
from jax import lax
from jax.experimental import pallas as pl
from jax.experimental.pallas import tpu as pltpu
from jax.experimental.pallas import tpu_sc as plsc

NN = 10000
EE = 320000
DD = 128
NC = 2
NS = 16
NW = NC * NS
EPT = EE // NW
CK = 80
NCH = EPT // CK
NROW = EE // CK
NPAD = 10240
RPT = NPAD // NS
RPT2 = NN // NS

_mesh = plsc.VectorSubcoreMesh(
    core_axis_name="c", subcore_axis_name="s", num_cores=NC, num_subcores=NS
)
_sc_params = pltpu.CompilerParams(
    use_tc_tiling_on_sc=False, needs_layout_passes=False
)


@functools.partial(
    pl.kernel,
    out_type=jax.ShapeDtypeStruct((NC, NPAD), jnp.float32),
    mesh=_mesh,
    compiler_params=_sc_params,
    scratch_types=[
        pltpu.VMEM((NCH, CK), jnp.int32),
        pltpu.VMEM((NCH, CK), jnp.float32),
        pltpu.VMEM((RPT,), jnp.float32),
        pltpu.VMEM_SHARED((NPAD,), jnp.float32),
    ],
)
def _deg_kernel(dst_hbm, w_hbm, out_hbm, dst_v, w_v, zer_v, acc_sh):
  cid = lax.axis_index("c")
  sid = lax.axis_index("s")
  wid = cid * NS + sid
  zvec = jnp.zeros((16,), jnp.float32)

  def _zero(i, carry):
    zer_v[pl.ds(i * 16, 16)] = zvec
    return carry

  lax.fori_loop(0, RPT // 16, _zero, 0)
  pltpu.sync_copy(zer_v, acc_sh.at[pl.ds(sid * RPT, RPT)])
  plsc.subcore_barrier()

  r0 = wid * NCH
  pltpu.sync_copy(dst_hbm.at[pl.ds(r0, NCH)], dst_v)
  pltpu.sync_copy(w_hbm.at[pl.ds(r0, NCH)], w_v)

  def _scatter(g, carry):
    pltpu.sync_copy(w_v.at[g], acc_sh.at[dst_v.at[g]], add=True)
    return carry

  lax.fori_loop(0, NCH, _scatter, 0)
  plsc.subcore_barrier()
  pltpu.sync_copy(
      acc_sh.at[pl.ds(sid * RPT, RPT)], out_hbm.at[cid, pl.ds(sid * RPT, RPT)]
  )


def _tc1_body(x_ref, w1_ref, degp2_ref, hs_ref, dinv_ref):
  h = jnp.dot(x_ref[...], w1_ref[...], preferred_element_type=jnp.float32)
  deg = degp2_ref[0] + degp2_ref[1] + 1.0
  dinv = jnp.where(deg > 0, lax.rsqrt(deg), 0.0)
  dinv_ref[...] = dinv
  hs_ref[...] = h * dinv


_TCB = 2000
def _tc1(x, w1, degp2):
  return pl.pallas_call(
      _tc1_body,
      grid=(NN // _TCB,),
      in_specs=[
          pl.BlockSpec((_TCB, DD), lambda i: (i, 0)),
          pl.BlockSpec((DD, DD), lambda i: (0, 0)),
          pl.BlockSpec((NC, _TCB, 1), lambda i: (0, i, 0)),
      ],
      out_specs=[
          pl.BlockSpec((_TCB, DD), lambda i: (i, 0)),
          pl.BlockSpec((_TCB, 1), lambda i: (i, 0)),
      ],
      out_shape=[
          jax.ShapeDtypeStruct((NN, DD), jnp.float32),
          jax.ShapeDtypeStruct((NN, 1), jnp.float32),
      ],
  )(x, w1, degp2)


GR = DD // 32


@functools.partial(
    pl.kernel,
    out_type=jax.ShapeDtypeStruct((NC, NN, DD), jnp.float32),
    mesh=_mesh,
    compiler_params=_sc_params,
    scratch_types=[
        pltpu.VMEM((NCH, CK), jnp.int32),
        pltpu.VMEM((NCH, CK), jnp.float32),
        pltpu.VMEM((CK, DD), jnp.bfloat16),
        pltpu.VMEM((CK, DD), jnp.bfloat16),
        pltpu.VMEM((CK, DD), jnp.bfloat16),
        pltpu.VMEM((CK, DD), jnp.float32),
        pltpu.VMEM((CK,), jnp.int32),
        pltpu.VMEM((CK,), jnp.int32),
        pltpu.VMEM((CK,), jnp.int32),
        pltpu.VMEM((CK,), jnp.int32),
        pltpu.VMEM_SHARED((NN, DD), jnp.float32),
        pltpu.SemaphoreType.DMA,
        pltpu.SemaphoreType.DMA,
        pltpu.SemaphoreType.DMA,
        pltpu.SemaphoreType.DMA,
    ],
)
def _conv_kernel(
    pk_hbm, w_hbm, hsb_hbm, out_hbm,
    pk_v, w_v, rb0, rb1, rb2, stg, si0, si1, si2, didx, acc_sh,
    gs0, gs1, gs2, ssem,
):
  cid = lax.axis_index("c")
  sid = lax.axis_index("s")
  wid = cid * NS + sid
  zvec = jnp.zeros((16,), jnp.float32)
  zivec = jnp.zeros((16,), jnp.int32)

  rbufs = (rb0, rb1, rb2)
  sibufs = (si0, si1, si2)
  gsems = (gs0, gs1, gs2)

  def _zero_rows(i, carry):
    for j in range(DD // 16):
      stg[i, pl.ds(j * 16, 16)] = zvec
    return carry

  lax.fori_loop(0, CK, _zero_rows, 0)

  def _zero_acc(k, carry):
    pltpu.sync_copy(stg, acc_sh.at[pl.ds(sid * RPT2 + k * CK, CK)])
    return carry

  lax.fori_loop(0, RPT2 // CK, _zero_acc, 0)
  pltpu.sync_copy(
      stg.at[pl.ds(0, RPT2 % CK)],
      acc_sh.at[pl.ds(sid * RPT2 + (RPT2 // CK) * CK, RPT2 % CK)],
  )
  plsc.subcore_barrier()

  r0 = wid * NCH
  pltpu.sync_copy(pk_hbm.at[pl.ds(r0, NCH)], pk_v)
  pltpu.sync_copy(w_hbm.at[pl.ds(r0, NCH)], w_v)

  def _unpack_src(g, sib):
    for i in range(CK // 16):
      sl = pl.ds(i * 16, 16)
      sib[sl] = pk_v[g, sl] & 16383

  def _gather(sib, rb, sem):
    pltpu.async_copy(hsb_hbm.at[sib], rb, sem)

  def _wait_gather(rb, sem):
    pltpu.make_async_copy(hsb_hbm.at[si0], rb, sem).wait()

  def _process(g, rb):
    for i in range(CK // 16):
      sl = pl.ds(i * 16, 16)
      didx[sl] = pk_v[g, sl] >> 14

    g16 = jnp.full((16,), g, jnp.int32)

    @plsc.parallel_loop(0, CK, step=1, unroll=2)
    def _scale(e):
      cs = plsc.load_gather(w_v, [g16, jnp.full((16,), e, jnp.int32)])
      for j in range(GR):
        v = plsc.bitcast(rb[e, pl.ds(32 * j, 32)], jnp.int32)
        lo = plsc.bitcast(v << 16, jnp.float32)
        hi = plsc.bitcast(v & jnp.int32(-65536), jnp.float32)
        stg[e, pl.ds(32 * j, 16)] = lo * cs
        stg[e, pl.ds(32 * j + 16, 16)] = hi * cs

  def _wait_scatter():
    pltpu.make_async_copy(stg, acc_sh.at[didx], ssem).wait()

  for i in range(CK // 16):
    didx[pl.ds(i * 16, 16)] = zivec
  pltpu.async_copy(stg, acc_sh.at[didx], ssem, add=True)

  for b in range(3):
    _unpack_src(b, sibufs[b])
    _gather(sibufs[b], rbufs[b], gsems[b])

  def _step(c, b):
    _wait_gather(rbufs[b], gsems[b])
    _wait_scatter()
    _process(c, rbufs[b])
    pltpu.async_copy(stg, acc_sh.at[didx], ssem, add=True)
    cn = jnp.minimum(c + 3, NCH - 1)
    _unpack_src(cn, sibufs[b])
    _gather(sibufs[b], rbufs[b], gsems[b])

  def _loop(it, carry):
    c = 3 * it
    _step(c, 0)
    _step(c + 1, 1)
    _step(c + 2, 2)
    return carry

  lax.fori_loop(0, NCH // 3, _loop, 0)

  def _tail(c, b):
    _wait_gather(rbufs[b], gsems[b])
    _wait_scatter()
    _process(c, rbufs[b])
    pltpu.async_copy(stg, acc_sh.at[didx], ssem, add=True)

  _tail(NCH - 2, 0)
  _tail(NCH - 1, 1)
  _wait_gather(rbufs[2], gsems[2])
  _wait_scatter()

  plsc.subcore_barrier()
  pltpu.sync_copy(
      acc_sh.at[pl.ds(sid * RPT2, RPT2)],
      out_hbm.at[cid, pl.ds(sid * RPT2, RPT2)],
  )


def _tc2_body(p_ref, hs_ref, dinv_ref, b1_ref, w2_ref, emb_ref, ts_ref):
  dinv = dinv_ref[...]
  out1 = dinv * (p_ref[0] + p_ref[1] + hs_ref[...]) + b1_ref[...]
  emb = jnp.where(out1 > 0, out1, jnp.exp(out1) - 1.0)
  emb_ref[...] = emb
  t = jnp.dot(emb, w2_ref[...], preferred_element_type=jnp.float32)
  ts_ref[...] = dinv * t


def _tc2(p, hs, dinv2d, b1, w2):
  return pl.pallas_call(
      _tc2_body,
      grid=(NN // _TCB,),
      in_specs=[
          pl.BlockSpec((NC, _TCB, DD), lambda i: (0, i, 0)),
          pl.BlockSpec((_TCB, DD), lambda i: (i, 0)),
          pl.BlockSpec((_TCB, 1), lambda i: (i, 0)),
          pl.BlockSpec((DD,), lambda i: (0,)),
          pl.BlockSpec((DD, 1), lambda i: (0, 0)),
      ],
      out_specs=[
          pl.BlockSpec((_TCB, DD), lambda i: (i, 0)),
          pl.BlockSpec((_TCB, 1), lambda i: (i, 0)),
      ],
      out_shape=[
          jax.ShapeDtypeStruct((NN, DD), jnp.float32),
          jax.ShapeDtypeStruct((NN, 1), jnp.float32),
      ],
  )(p, hs, dinv2d, b1, w2)


@functools.partial(
    pl.kernel,
    out_type=jax.ShapeDtypeStruct((NC, NPAD), jnp.float32),
    mesh=_mesh,
    compiler_params=_sc_params,
    scratch_types=[
        pltpu.VMEM((NCH, CK), jnp.int32),
        pltpu.VMEM((NCH, CK), jnp.int32),
        pltpu.VMEM((NCH, CK), jnp.float32),
        pltpu.VMEM((NN,), jnp.float32),
        pltpu.VMEM((CK,), jnp.float32),
        pltpu.VMEM((CK,), jnp.float32),
        pltpu.VMEM((RPT,), jnp.float32),
        pltpu.VMEM_SHARED((NPAD,), jnp.float32),
        pltpu.SemaphoreType.DMA,
        pltpu.SemaphoreType.DMA,
    ],
)
def _l2_kernel(
    src_hbm, dst_hbm, w_hbm, ts_hbm, out_hbm,
    src_v, dst_v, w_v, ts_v, vals_a, vals_b, zer_v, acc_sh, ssa, ssb,
):
  cid = lax.axis_index("c")
  sid = lax.axis_index("s")
  wid = cid * NS + sid
  zvec = jnp.zeros((16,), jnp.float32)

  def _zero(i, carry):
    zer_v[pl.ds(i * 16, 16)] = zvec
    return carry

  lax.fori_loop(0, RPT // 16, _zero, 0)
  pltpu.sync_copy(zer_v, acc_sh.at[pl.ds(sid * RPT, RPT)])
  plsc.subcore_barrier()

  r0 = wid * NCH
  pltpu.sync_copy(src_hbm.at[pl.ds(r0, NCH)], src_v)
  pltpu.sync_copy(dst_hbm.at[pl.ds(r0, NCH)], dst_v)
  pltpu.sync_copy(w_hbm.at[pl.ds(r0, NCH)], w_v)
  pltpu.sync_copy(ts_hbm, ts_v)

  def _vals(g, vv):
    for i in range(CK // 16):
      sl = pl.ds(i * 16, 16)
      s16 = src_v[g, sl]
      ts = plsc.load_gather(ts_v, [s16])
      vv[sl] = w_v[g, sl] * ts

  def _fire(g, vv, sem):
    pltpu.async_copy(vv, acc_sh.at[dst_v.at[g]], sem, add=True)

  def _drain(vv, sem):
    pltpu.make_async_copy(vv, acc_sh.at[dst_v.at[0]], sem).wait()

  _vals(0, vals_a)
  _fire(0, vals_a, ssa)
  _vals(1, vals_b)
  _fire(1, vals_b, ssb)

  def _chunk(it, carry):
    g0 = 2 * it
    _drain(vals_a, ssa)
    _vals(g0, vals_a)
    _fire(g0, vals_a, ssa)
    _drain(vals_b, ssb)
    _vals(g0 + 1, vals_b)
    _fire(g0 + 1, vals_b, ssb)
    return carry

  lax.fori_loop(1, NCH // 2, _chunk, 0)

  _drain(vals_a, ssa)
  _vals(NCH - 1, vals_a)
  _fire(NCH - 1, vals_a, ssa)
  _drain(vals_a, ssa)
  _drain(vals_b, ssb)
  plsc.subcore_barrier()
  pltpu.sync_copy(
      acc_sh.at[pl.ds(sid * RPT, RPT)], out_hbm.at[cid, pl.ds(sid * RPT, RPT)]
  )


def _tc3_body(q_ref, ts1_ref, dinv1_ref, b2_ref, out_ref):
  d = dinv1_ref[...]
  z = d * (q_ref[0, 0:NN] + q_ref[1, 0:NN] + ts1_ref[...]) + b2_ref[...]
  out_ref[...] = 1.0 / (1.0 + jnp.exp(-z))


def _tc3(q, ts1, dinv1, b2):
  return pl.pallas_call(
      _tc3_body,
      out_shape=jax.ShapeDtypeStruct((NN,), jnp.float32),
  )(q, ts1, dinv1, b2)


_COLPERM = []
for _g in range(DD // 32):
  for _m in range(16):
    _COLPERM.append(32 * _g + _m)
    _COLPERM.append(32 * _g + 16 + _m)
_COLPERM = tuple(_COLPERM)


def kernel(x, edge_index, edge_attr, W1, b1, W2, b2):
  src2 = edge_index[0].reshape(NROW, CK)
  dst2 = edge_index[1].reshape(NROW, CK)
  w2d = edge_attr.reshape(NROW, CK)
  pk2 = src2 | (dst2 << 14)

  degp = _deg_kernel(dst2, w2d)
  degp2 = degp[:, :NN].reshape(NC, NN, 1)
  hs, dinv2d = _tc1(x, W1, degp2)
  hsb = hs[:, _COLPERM].astype(jnp.bfloat16)
  p = _conv_kernel(pk2, w2d, hsb)
  emb, ts = _tc2(p, hs, dinv2d, b1, W2)
  q = _l2_kernel(src2, dst2, w2d, ts.reshape(NN))
  out1 = _tc3(q, ts.reshape(NN), dinv2d.reshape(NN), b2)
  return (out1.reshape(NN, 1), emb)
